# baseline (speedup 1.0000x reference)
import functools

import jax
import jax.numpy as jnp
from jax.experimental import pallas as pl
from jax.experimental.pallas import tpu as pltpu

EPS = 1e-6


def _moe_kernel(xp_ref, lnpre_ref, rw_ref, sg_ref, su_ref, sd_ref,
                eg_ref, eu_ref, ed_ref, ls_ref, lnpost_ref,
                out_ref, fn_ref, i1_ref, i2_ref, w1_ref, acc_ref, *, n_experts):
    e = pl.program_id(0)

    @pl.when(e == 0)
    def _prologue():
        h = xp_ref[...]
        var = jnp.mean(h * h, axis=-1, keepdims=True)
        fn = (h * jax.lax.rsqrt(var + EPS)) * lnpre_ref[...]
        fn_ref[...] = fn.astype(jnp.bfloat16)
        fnb = fn_ref[...]
        g = jnp.dot(fnb, sg_ref[...].astype(jnp.bfloat16),
                    preferred_element_type=jnp.float32)
        u = jnp.dot(fnb, su_ref[...].astype(jnp.bfloat16),
                    preferred_element_type=jnp.float32)
        acc_ref[...] = jnp.dot((jax.nn.silu(g) * u).astype(jnp.bfloat16),
                               sd_ref[...].astype(jnp.bfloat16),
                               preferred_element_type=jnp.float32)
        logits = jnp.dot(fn, rw_ref[...], preferred_element_type=jnp.float32)
        n, ne = logits.shape
        iota = jax.lax.broadcasted_iota(jnp.int32, (n, ne), 1)
        m1 = jnp.max(logits, axis=-1, keepdims=True)
        i1 = jnp.min(jnp.where(logits == m1, iota, ne), axis=-1, keepdims=True)
        masked = jnp.where(iota == i1, -jnp.inf, logits)
        m2 = jnp.max(masked, axis=-1, keepdims=True)
        i2 = jnp.min(jnp.where(masked == m2, iota, ne), axis=-1, keepdims=True)
        i1_ref[...] = i1
        i2_ref[...] = i2
        w1_ref[...] = jax.nn.sigmoid(m1 - m2)

    fnb = fn_ref[...]
    g = jnp.dot(fnb, eg_ref[0].astype(jnp.bfloat16),
                preferred_element_type=jnp.float32)
    u = jnp.dot(fnb, eu_ref[0].astype(jnp.bfloat16),
                preferred_element_type=jnp.float32)
    hmid = jax.nn.silu(g) * u
    w1 = w1_ref[...]
    ce = (jnp.where(i1_ref[...] == e, w1, 0.0)
          + jnp.where(i2_ref[...] == e, 1.0 - w1, 0.0))
    acc_ref[...] += jnp.dot((hmid * ce).astype(jnp.bfloat16),
                            ed_ref[0].astype(jnp.bfloat16),
                            preferred_element_type=jnp.float32)

    @pl.when(e == n_experts - 1)
    def _epilogue():
        a = acc_ref[...] * ls_ref[...]
        var = jnp.mean(a * a, axis=-1, keepdims=True)
        out_ref[...] = (a * jax.lax.rsqrt(var + EPS)) * lnpost_ref[...]


@jax.jit
def kernel(x, ln_pre_w, router_w, sh_gate, sh_up, sh_down, eg, eu, ed,
           layer_scale, ln_post_w):
    b, t, d = x.shape
    in_dim = ln_pre_w.shape[0]
    k_pool = in_dim // d
    t2 = (t // k_pool) * k_pool
    n = b * (t2 // k_pool)
    n_experts = router_w.shape[1]
    hid = sh_gate.shape[1]
    out_dim = sh_down.shape[1]

    xp = x[:, :t2, :].reshape(n, in_dim)
    full = lambda shape: pl.BlockSpec(shape, lambda e: (0,) * len(shape))

    out = pl.pallas_call(
        functools.partial(_moe_kernel, n_experts=n_experts),
        grid=(n_experts,),
        in_specs=[
            full((n, in_dim)),
            full((1, in_dim)),
            full((in_dim, n_experts)),
            full((in_dim, hid)),
            full((in_dim, hid)),
            full((hid, out_dim)),
            pl.BlockSpec((1, in_dim, hid), lambda e: (e, 0, 0)),
            pl.BlockSpec((1, in_dim, hid), lambda e: (e, 0, 0)),
            pl.BlockSpec((1, hid, out_dim), lambda e: (e, 0, 0)),
            full((1, out_dim)),
            full((1, out_dim)),
        ],
        out_specs=full((n, out_dim)),
        out_shape=jax.ShapeDtypeStruct((n, out_dim), jnp.float32),
        scratch_shapes=[
            pltpu.VMEM((n, in_dim), jnp.bfloat16),
            pltpu.VMEM((n, 1), jnp.int32),
            pltpu.VMEM((n, 1), jnp.int32),
            pltpu.VMEM((n, 1), jnp.float32),
            pltpu.VMEM((n, out_dim), jnp.float32),
        ],
    )(xp, ln_pre_w.reshape(1, in_dim), router_w, sh_gate, sh_up, sh_down,
      eg, eu, ed, layer_scale.reshape(1, out_dim), ln_post_w.reshape(1, out_dim))
    return out.reshape(b, t2 // k_pool, out_dim)


# speedup vs baseline: 1.1947x; 1.1947x over previous
import functools

import jax
import jax.numpy as jnp
from jax.experimental import pallas as pl
from jax.experimental.pallas import tpu as pltpu

EPS = 1e-6


def _moe_kernel(xp_ref, lnpre_ref, rw_ref, sg_ref, su_ref, sd_ref,
                eg_ref, eu_ref, ed_ref, ls_ref, lnpost_ref,
                out_ref, fn_ref, i1_ref, i2_ref, w1_ref, acc_ref,
                egb_ref, eub_ref, edb_ref, sg_sem, su_sem, sd_sem,
                *, n_experts):
    e = pl.program_id(0)

    def start(idx, slot):
        pltpu.make_async_copy(eg_ref.at[idx], egb_ref.at[slot],
                              sg_sem.at[slot]).start()
        pltpu.make_async_copy(eu_ref.at[idx], eub_ref.at[slot],
                              su_sem.at[slot]).start()
        pltpu.make_async_copy(ed_ref.at[idx], edb_ref.at[slot],
                              sd_sem.at[slot]).start()

    @pl.when(e == 0)
    def _prologue():
        start(0, 0)
        start(1, 1)
        h = xp_ref[...]
        var = jnp.mean(h * h, axis=-1, keepdims=True)
        fn = (h * jax.lax.rsqrt(var + EPS)) * lnpre_ref[...]
        fn_ref[...] = fn.astype(jnp.bfloat16)
        logits = jnp.dot(fn, rw_ref[...], preferred_element_type=jnp.float32)
        n, ne = logits.shape
        iota = jax.lax.broadcasted_iota(jnp.int32, (n, ne), 1)
        m1 = jnp.max(logits, axis=-1, keepdims=True)
        i1 = jnp.min(jnp.where(logits == m1, iota, ne), axis=-1, keepdims=True)
        masked = jnp.where(iota == i1, -jnp.inf, logits)
        m2 = jnp.max(masked, axis=-1, keepdims=True)
        i2 = jnp.min(jnp.where(masked == m2, iota, ne), axis=-1, keepdims=True)
        i1_ref[...] = i1
        i2_ref[...] = i2
        w1_ref[...] = jax.nn.sigmoid(m1 - m2)
        fnb = fn_ref[...]
        g = jnp.dot(fnb, sg_ref[...].astype(jnp.bfloat16),
                    preferred_element_type=jnp.float32)
        u = jnp.dot(fnb, su_ref[...].astype(jnp.bfloat16),
                    preferred_element_type=jnp.float32)
        acc_ref[...] = jnp.dot((jax.nn.silu(g) * u).astype(jnp.bfloat16),
                               sd_ref[...].astype(jnp.bfloat16),
                               preferred_element_type=jnp.float32)

    slot = jax.lax.rem(e, 2)
    pltpu.make_async_copy(eg_ref.at[e], egb_ref.at[slot], sg_sem.at[slot]).wait()
    pltpu.make_async_copy(eu_ref.at[e], eub_ref.at[slot], su_sem.at[slot]).wait()
    pltpu.make_async_copy(ed_ref.at[e], edb_ref.at[slot], sd_sem.at[slot]).wait()

    fnb = fn_ref[...]
    g = jnp.dot(fnb, egb_ref[slot].astype(jnp.bfloat16),
                preferred_element_type=jnp.float32)
    u = jnp.dot(fnb, eub_ref[slot].astype(jnp.bfloat16),
                preferred_element_type=jnp.float32)
    hmid = jax.nn.silu(g) * u
    w1 = w1_ref[...]
    ce = (jnp.where(i1_ref[...] == e, w1, 0.0)
          + jnp.where(i2_ref[...] == e, 1.0 - w1, 0.0))
    acc_ref[...] += jnp.dot((hmid * ce).astype(jnp.bfloat16),
                            edb_ref[slot].astype(jnp.bfloat16),
                            preferred_element_type=jnp.float32)

    @pl.when(e + 2 < n_experts)
    def _prefetch_next():
        start(e + 2, slot)

    @pl.when(e == n_experts - 1)
    def _epilogue():
        a = acc_ref[...] * ls_ref[...]
        var = jnp.mean(a * a, axis=-1, keepdims=True)
        out_ref[...] = (a * jax.lax.rsqrt(var + EPS)) * lnpost_ref[...]


@jax.jit
def kernel(x, ln_pre_w, router_w, sh_gate, sh_up, sh_down, eg, eu, ed,
           layer_scale, ln_post_w):
    b, t, d = x.shape
    in_dim = ln_pre_w.shape[0]
    k_pool = in_dim // d
    t2 = (t // k_pool) * k_pool
    n = b * (t2 // k_pool)
    n_experts = router_w.shape[1]
    hid = sh_gate.shape[1]
    out_dim = sh_down.shape[1]

    xp = x[:, :t2, :].reshape(n, in_dim)
    full = lambda shape: pl.BlockSpec(shape, lambda e: (0,) * len(shape))
    anyspec = pl.BlockSpec(memory_space=pl.ANY)

    out = pl.pallas_call(
        functools.partial(_moe_kernel, n_experts=n_experts),
        grid=(n_experts,),
        in_specs=[
            full((n, in_dim)),
            full((1, in_dim)),
            full((in_dim, n_experts)),
            full((in_dim, hid)),
            full((in_dim, hid)),
            full((hid, out_dim)),
            anyspec,
            anyspec,
            anyspec,
            full((1, out_dim)),
            full((1, out_dim)),
        ],
        out_specs=full((n, out_dim)),
        out_shape=jax.ShapeDtypeStruct((n, out_dim), jnp.float32),
        scratch_shapes=[
            pltpu.VMEM((n, in_dim), jnp.bfloat16),
            pltpu.VMEM((n, 1), jnp.int32),
            pltpu.VMEM((n, 1), jnp.int32),
            pltpu.VMEM((n, 1), jnp.float32),
            pltpu.VMEM((n, out_dim), jnp.float32),
            pltpu.VMEM((2, in_dim, hid), jnp.float32),
            pltpu.VMEM((2, in_dim, hid), jnp.float32),
            pltpu.VMEM((2, hid, out_dim), jnp.float32),
            pltpu.SemaphoreType.DMA((2,)),
            pltpu.SemaphoreType.DMA((2,)),
            pltpu.SemaphoreType.DMA((2,)),
        ],
    )(xp, ln_pre_w.reshape(1, in_dim), router_w, sh_gate, sh_up, sh_down,
      eg, eu, ed, layer_scale.reshape(1, out_dim), ln_post_w.reshape(1, out_dim))
    return out.reshape(b, t2 // k_pool, out_dim)


# speedup vs baseline: 1.2070x; 1.0104x over previous
import functools

import jax
import jax.numpy as jnp
from jax.experimental import pallas as pl
from jax.experimental.pallas import tpu as pltpu

EPS = 1e-6


def _moe_kernel(xp_ref, lnpre_ref, rw_ref, sg_ref, su_ref, sd_ref,
                eg_ref, eu_ref, ed_ref, ls_ref, lnpost_ref,
                out_ref, fn_ref, i1_ref, i2_ref, w1_ref, acc_ref,
                egb_ref, eub_ref, edb_ref, sg_sem, su_sem, sd_sem,
                *, n_experts):
    e = pl.program_id(0)

    def _copies(idx, slot):
        h2 = eg_ref.shape[2] // 2
        i2h = ed_ref.shape[1]
        return [
            pltpu.make_async_copy(eg_ref.at[idx, :, :h2],
                                  egb_ref.at[slot, :, :h2], sg_sem.at[slot, 0]),
            pltpu.make_async_copy(eg_ref.at[idx, :, h2:],
                                  egb_ref.at[slot, :, h2:], sg_sem.at[slot, 1]),
            pltpu.make_async_copy(eu_ref.at[idx, :, :h2],
                                  eub_ref.at[slot, :, :h2], su_sem.at[slot, 0]),
            pltpu.make_async_copy(eu_ref.at[idx, :, h2:],
                                  eub_ref.at[slot, :, h2:], su_sem.at[slot, 1]),
            pltpu.make_async_copy(ed_ref.at[idx, :i2h // 2],
                                  edb_ref.at[slot, :i2h // 2], sd_sem.at[slot, 0]),
            pltpu.make_async_copy(ed_ref.at[idx, i2h // 2:],
                                  edb_ref.at[slot, i2h // 2:], sd_sem.at[slot, 1]),
        ]

    def start(idx, slot):
        for c in _copies(idx, slot):
            c.start()

    @pl.when(e == 0)
    def _prologue():
        start(0, 0)
        start(1, 1)
        h = xp_ref[...]
        var = jnp.mean(h * h, axis=-1, keepdims=True)
        fn = (h * jax.lax.rsqrt(var + EPS)) * lnpre_ref[...]
        fn_ref[...] = fn.astype(jnp.bfloat16)
        logits = jnp.dot(fn, rw_ref[...], preferred_element_type=jnp.float32)
        n, ne = logits.shape
        iota = jax.lax.broadcasted_iota(jnp.int32, (n, ne), 1)
        m1 = jnp.max(logits, axis=-1, keepdims=True)
        i1 = jnp.min(jnp.where(logits == m1, iota, ne), axis=-1, keepdims=True)
        masked = jnp.where(iota == i1, -jnp.inf, logits)
        m2 = jnp.max(masked, axis=-1, keepdims=True)
        i2 = jnp.min(jnp.where(masked == m2, iota, ne), axis=-1, keepdims=True)
        i1_ref[...] = i1
        i2_ref[...] = i2
        w1_ref[...] = jax.nn.sigmoid(m1 - m2)
        fnb = fn_ref[...]
        g = jnp.dot(fnb, sg_ref[...].astype(jnp.bfloat16),
                    preferred_element_type=jnp.float32)
        u = jnp.dot(fnb, su_ref[...].astype(jnp.bfloat16),
                    preferred_element_type=jnp.float32)
        acc_ref[...] = jnp.dot((jax.nn.silu(g) * u).astype(jnp.bfloat16),
                               sd_ref[...].astype(jnp.bfloat16),
                               preferred_element_type=jnp.float32)

    slot = jax.lax.rem(e, 2)
    for c in _copies(e, slot):
        c.wait()

    fnb = fn_ref[...]
    g = jnp.dot(fnb, egb_ref[slot].astype(jnp.bfloat16),
                preferred_element_type=jnp.float32)
    u = jnp.dot(fnb, eub_ref[slot].astype(jnp.bfloat16),
                preferred_element_type=jnp.float32)
    hmid = jax.nn.silu(g) * u
    w1 = w1_ref[...]
    ce = (jnp.where(i1_ref[...] == e, w1, 0.0)
          + jnp.where(i2_ref[...] == e, 1.0 - w1, 0.0))
    acc_ref[...] += jnp.dot((hmid * ce).astype(jnp.bfloat16),
                            edb_ref[slot].astype(jnp.bfloat16),
                            preferred_element_type=jnp.float32)

    @pl.when(e + 2 < n_experts)
    def _prefetch_next():
        start(e + 2, slot)

    @pl.when(e == n_experts - 1)
    def _epilogue():
        a = acc_ref[...] * ls_ref[...]
        var = jnp.mean(a * a, axis=-1, keepdims=True)
        out_ref[...] = (a * jax.lax.rsqrt(var + EPS)) * lnpost_ref[...]


@jax.jit
def kernel(x, ln_pre_w, router_w, sh_gate, sh_up, sh_down, eg, eu, ed,
           layer_scale, ln_post_w):
    b, t, d = x.shape
    in_dim = ln_pre_w.shape[0]
    k_pool = in_dim // d
    t2 = (t // k_pool) * k_pool
    n = b * (t2 // k_pool)
    n_experts = router_w.shape[1]
    hid = sh_gate.shape[1]
    out_dim = sh_down.shape[1]

    xp = x[:, :t2, :].reshape(n, in_dim)
    full = lambda shape: pl.BlockSpec(shape, lambda e: (0,) * len(shape))
    anyspec = pl.BlockSpec(memory_space=pl.ANY)

    out = pl.pallas_call(
        functools.partial(_moe_kernel, n_experts=n_experts),
        grid=(n_experts,),
        in_specs=[
            full((n, in_dim)),
            full((1, in_dim)),
            full((in_dim, n_experts)),
            full((in_dim, hid)),
            full((in_dim, hid)),
            full((hid, out_dim)),
            anyspec,
            anyspec,
            anyspec,
            full((1, out_dim)),
            full((1, out_dim)),
        ],
        out_specs=full((n, out_dim)),
        out_shape=jax.ShapeDtypeStruct((n, out_dim), jnp.float32),
        scratch_shapes=[
            pltpu.VMEM((n, in_dim), jnp.bfloat16),
            pltpu.VMEM((n, 1), jnp.int32),
            pltpu.VMEM((n, 1), jnp.int32),
            pltpu.VMEM((n, 1), jnp.float32),
            pltpu.VMEM((n, out_dim), jnp.float32),
            pltpu.VMEM((2, in_dim, hid), jnp.float32),
            pltpu.VMEM((2, in_dim, hid), jnp.float32),
            pltpu.VMEM((2, hid, out_dim), jnp.float32),
            pltpu.SemaphoreType.DMA((2, 2)),
            pltpu.SemaphoreType.DMA((2, 2)),
            pltpu.SemaphoreType.DMA((2, 2)),
        ],
    )(xp, ln_pre_w.reshape(1, in_dim), router_w, sh_gate, sh_up, sh_down,
      eg, eu, ed, layer_scale.reshape(1, out_dim), ln_post_w.reshape(1, out_dim))
    return out.reshape(b, t2 // k_pool, out_dim)
